# baseline (speedup 1.0000x reference)
import functools

import jax
import jax.numpy as jnp
from jax import lax
from jax.experimental import pallas as pl
from jax.experimental.pallas import tpu as pltpu
from jax.experimental.pallas import tpu_sc as plsc

B = 16384
EMB = 16
NC = 2
NS = 16
NW = NC * NS
BPW = B // NW
CHUNK = 128
NCH = BPW // CHUNK

@functools.lru_cache(maxsize=None)
def _make_sc_gather():
    mesh = plsc.VectorSubcoreMesh(
        core_axis_name="c", subcore_axis_name="s", num_cores=NC, num_subcores=NS
    )

    @functools.partial(
        pl.kernel,
        out_type=[
            jax.ShapeDtypeStruct((B, EMB), jnp.float32),
            jax.ShapeDtypeStruct((B, EMB), jnp.float32),
        ],
        mesh=mesh,
        scratch_types=[
            pltpu.VMEM((NCH, CHUNK), jnp.int32),
            pltpu.VMEM((NCH, CHUNK), jnp.int32),
            pltpu.VMEM((BPW, EMB), jnp.float32),
            pltpu.VMEM((BPW, EMB), jnp.float32),
            pltpu.SemaphoreType.DMA,
        ],
        compiler_params=pltpu.CompilerParams(use_tc_tiling_on_sc=False),
    )
    def sc_gather(row_tab, col_tab, ridx, cidx, row_out, col_out,
                  ridx_v, cidx_v, rrows_v, crows_v, sem):
        wid = lax.axis_index("s") * NC + lax.axis_index("c")
        base = wid * BPW
        pltpu.sync_copy(ridx.at[wid], ridx_v)
        pltpu.sync_copy(cidx.at[wid], cidx_v)
        copies = []
        for j in range(NCH):
            copies.append(pltpu.async_copy(
                row_tab.at[ridx_v.at[j]], rrows_v.at[pl.ds(j * CHUNK, CHUNK)], sem))
            copies.append(pltpu.async_copy(
                col_tab.at[cidx_v.at[j]], crows_v.at[pl.ds(j * CHUNK, CHUNK)], sem))
        for c in copies:
            c.wait()
        pltpu.sync_copy(rrows_v, row_out.at[pl.ds(base, BPW)])
        pltpu.sync_copy(crows_v, col_out.at[pl.ds(base, BPW)])

    return sc_gather


BLK = 2048


def _mlp_body(x_ref, re_ref, ce_ref, W1_ref, b1_ref, g1_ref, be1_ref,
              W2_ref, b2_ref, g2_ref, be2_ref, W3_ref, b3_ref, o_ref):
    x = x_ref[...]
    re = re_ref[...]
    ce = ce_ref[...]
    W1 = W1_ref[...]
    h = (x[:, 0:1] * W1[0:1, :] + x[:, 1:2] * W1[1:2, :]
         + jnp.dot(re, W1[2:2 + EMB, :], preferred_element_type=jnp.float32,
                   precision=lax.Precision.HIGHEST)
         + jnp.dot(ce, W1[2 + EMB:, :], preferred_element_type=jnp.float32,
                   precision=lax.Precision.HIGHEST)
         + b1_ref[...])
    h = _layernorm_gelu(h, g1_ref[...], be1_ref[...])
    h = jnp.dot(h, W2_ref[...], preferred_element_type=jnp.float32,
                precision=lax.Precision.HIGHEST) + b2_ref[...]
    h = _layernorm_gelu(h, g2_ref[...], be2_ref[...])
    o_ref[...] = jnp.sum(h * W3_ref[...], axis=-1, keepdims=True) + b3_ref[...]


def _layernorm_gelu(h, g, b, eps=1e-5):
    mu = jnp.mean(h, axis=-1, keepdims=True)
    var = jnp.mean((h - mu) ** 2, axis=-1, keepdims=True)
    h = (h - mu) / jnp.sqrt(var + eps) * g + b
    return h * 0.5 * (1.0 + lax.erf(h * (2.0 ** -0.5)))


def kernel(x, row_idx, col_idx, row_table, col_table,
           W1, b1, g1, be1, W2, b2, g2, be2, W3, b3):
    ridx = row_idx.astype(jnp.int32).reshape(NW, NCH, CHUNK)
    cidx = col_idx.astype(jnp.int32).reshape(NW, NCH, CHUNK)
    row_emb, col_emb = _make_sc_gather()(row_table, col_table, ridx, cidx)

    grid = (B // BLK,)
    full = lambda i: (0, 0)
    batch = lambda i: (i, 0)
    out = pl.pallas_call(
        _mlp_body,
        grid=grid,
        in_specs=[
            pl.BlockSpec((BLK, 2), batch),
            pl.BlockSpec((BLK, EMB), batch),
            pl.BlockSpec((BLK, EMB), batch),
            pl.BlockSpec((2 + 2 * EMB, 32), full),
            pl.BlockSpec((1, 32), full),
            pl.BlockSpec((1, 32), full),
            pl.BlockSpec((1, 32), full),
            pl.BlockSpec((32, 16), full),
            pl.BlockSpec((1, 16), full),
            pl.BlockSpec((1, 16), full),
            pl.BlockSpec((1, 16), full),
            pl.BlockSpec((1, 16), full),
            pl.BlockSpec((1, 1), full),
        ],
        out_specs=pl.BlockSpec((BLK, 1), batch),
        out_shape=jax.ShapeDtypeStruct((B, 1), jnp.float32),
    )(x, row_emb, col_emb, W1,
      b1.reshape(1, 32), g1.reshape(1, 32), be1.reshape(1, 32),
      W2, b2.reshape(1, 16), g2.reshape(1, 16), be2.reshape(1, 16),
      W3.reshape(1, 16), b3.reshape(1, 1))
    return out
